# baseline (speedup 1.0000x reference)
import functools

import jax
import jax.numpy as jnp
from jax import lax
from jax.experimental import pallas as pl
from jax.experimental.pallas import tpu as pltpu
from jax.experimental.pallas import tpu_sc as plsc

N = 10000
E = 320000
D = 128

NC = 2
NS = 16
L = 16
NW = NC * NS
EPW = E // NW
C = 80
NCHUNK = EPW // C
NPAIR = NCHUNK // 2
RPT = 624
TAIL = N - NS * RPT


_TC_BLK = 400


def _tc_pre_body(x_ref, ws_ref, wd_ref, wn_ref, bm_ref, bo_ref,
                 p_ref, q_ref, r_ref):
    xb = x_ref[...]
    p_ref[...] = jnp.dot(xb, ws_ref[...],
                         preferred_element_type=jnp.float32) + bm_ref[...]
    q_ref[...] = jnp.dot(xb, wd_ref[...], preferred_element_type=jnp.float32)
    r_ref[...] = jnp.dot(xb, wn_ref[...],
                         preferred_element_type=jnp.float32) + bo_ref[...]


def _tc_pre(x, w_src, w_dst, w_node, b_msg, b_out):
    row_spec = pl.BlockSpec((_TC_BLK, D), lambda i: (i, 0))
    full_spec = pl.BlockSpec((D, D), lambda i: (0, 0))
    bias_spec = pl.BlockSpec((1, D), lambda i: (0, 0))
    out_sds = jax.ShapeDtypeStruct((N, D), jnp.float32)
    return pl.pallas_call(
        _tc_pre_body,
        grid=(N // _TC_BLK,),
        in_specs=[row_spec, full_spec, full_spec, full_spec,
                  bias_spec, bias_spec],
        out_specs=[row_spec, row_spec, row_spec],
        out_shape=[out_sds, out_sds, out_sds],
    )(x, w_src, w_dst, w_node, b_msg, b_out)


def _sc_edge_body(p_hbm, q_hbm, idx_hbm, zero_hbm, out_hbm,
                  sidi0, sidi1, si0, si1, di0, di1, pbuf0, pbuf1,
                  qbuf0, qbuf1,
                  agg_sh, sem_p0, sem_p1, sem_q0, sem_q1, sem_s0, sem_s1,
                  sem_i0, sem_i1):
    c = lax.axis_index("c")
    s = lax.axis_index("s")
    wid = s * NC + c

    pltpu.sync_copy(zero_hbm.at[pl.ds(s * RPT, RPT)],
                    agg_sh.at[pl.ds(s * RPT, RPT)])

    @pl.when(s == NS - 1)
    def _zero_tail():
        pltpu.sync_copy(zero_hbm.at[pl.ds(NS * RPT, TAIL)],
                        agg_sh.at[pl.ds(NS * RPT, TAIL)])

    plsc.subcore_barrier()

    bufs = ((sidi0, si0, di0, pbuf0, qbuf0, sem_p0, sem_q0, sem_s0, sem_i0),
            (sidi1, si1, di1, pbuf1, qbuf1, sem_p1, sem_q1, sem_s1, sem_i1))

    def idx_copy(off, b):
        sidi, _, _, _, _, _, _, _, smi = bufs[b]
        base = (wid * NCHUNK + off) * 2 * C
        pltpu.async_copy(idx_hbm.at[pl.ds(base, 2 * C)], sidi, smi)

    def wait_idx(off, b):
        sidi, _, _, _, _, _, _, _, smi = bufs[b]
        base = (wid * NCHUNK + off) * 2 * C
        pltpu.make_async_copy(idx_hbm.at[pl.ds(base, 2 * C)], sidi, smi).wait()

    def gather(b):
        sidi, si, di, pb, qb, sp, sq, _, _ = bufs[b]
        for j in range(C // L):
            si[pl.ds(j * L, L)] = sidi[pl.ds(j * L, L)]
            di[pl.ds(j * L, L)] = sidi[pl.ds(C + j * L, L)]
        pltpu.async_copy(p_hbm.at[si], pb, sp)
        pltpu.async_copy(q_hbm.at[di], qb, sq)

    def wait_gather(b):
        _, si, di, pb, qb, sp, sq, _, _ = bufs[b]
        pltpu.make_async_copy(p_hbm.at[si], pb, sp).wait()
        pltpu.make_async_copy(q_hbm.at[di], qb, sq).wait()

    def scatter(b):
        _, _, di, pb, _, _, _, ss, _ = bufs[b]
        pltpu.async_copy(pb, agg_sh.at[di], ss, add=True)

    def wait_scatter(b):
        _, _, di, pb, _, _, _, ss, _ = bufs[b]
        pltpu.make_async_copy(pb, agg_sh.at[di], ss).wait()

    def compute(b):
        _, _, _, pb, qb, _, _, _, _ = bufs[b]

        def row(e, cr):
            for j in range(D // L):
                sl = pl.ds(j * L, L)
                pb[e, sl] = jnp.maximum(pb[e, sl] + qb[e, sl], 0.0)
            return cr

        lax.fori_loop(0, C, row, 0)

    idx_copy(0, 0)
    idx_copy(1, 1)
    wait_idx(0, 0)
    gather(0)

    def pair(g, carry):
        off0 = 2 * g

        wait_gather(0)

        @pl.when(g > 0)
        def _drain1():
            wait_scatter(1)

        wait_idx(off0 + 1, 1)
        gather(1)
        compute(0)
        scatter(0)
        idx_copy(off0 + 2, 0)

        wait_gather(1)
        wait_scatter(0)

        @pl.when(g < NPAIR - 1)
        def _next0():
            wait_idx(off0 + 2, 0)
            gather(0)

        compute(1)
        scatter(1)

        @pl.when(g < NPAIR - 1)
        def _prefetch1():
            idx_copy(off0 + 3, 1)

        return carry

    lax.fori_loop(0, NPAIR, pair, 0)

    wait_idx(NCHUNK - 1, 0)
    gather(0)
    wait_gather(0)
    wait_scatter(1)
    compute(0)
    scatter(0)
    wait_scatter(0)
    plsc.subcore_barrier()

    pltpu.sync_copy(agg_sh.at[pl.ds(s * RPT, RPT)],
                    out_hbm.at[c, pl.ds(s * RPT, RPT)])

    @pl.when(s == NS - 1)
    def _copy_tail():
        pltpu.sync_copy(agg_sh.at[pl.ds(NS * RPT, TAIL)],
                        out_hbm.at[c, pl.ds(NS * RPT, TAIL)])


_sc_edge = functools.partial(
    pl.kernel,
    out_type=jax.ShapeDtypeStruct((NC, N, D), jnp.float32),
    mesh=plsc.VectorSubcoreMesh(core_axis_name="c", subcore_axis_name="s"),
    scratch_types=(
        [pltpu.VMEM((2 * C,), jnp.int32)] * 2
        + [pltpu.VMEM((C,), jnp.int32)] * 4
        + [pltpu.VMEM((C, D), jnp.float32)] * 4
        + [pltpu.VMEM_SHARED((N, D), jnp.float32)]
        + [pltpu.SemaphoreType.DMA] * 8
    ),
)(_sc_edge_body)


def _tc_post_body(r_ref, a0_ref, a1_ref, wa_ref, o_ref):
    agg = a0_ref[...] + a1_ref[...]
    o_ref[...] = jnp.maximum(
        r_ref[...] + jnp.dot(agg, wa_ref[...],
                             preferred_element_type=jnp.float32), 0.0)


def _tc_post(r, a0, a1, w_agg):
    row_spec = pl.BlockSpec((_TC_BLK, D), lambda i: (i, 0))
    full_spec = pl.BlockSpec((D, D), lambda i: (0, 0))
    return pl.pallas_call(
        _tc_post_body,
        grid=(N // _TC_BLK,),
        in_specs=[row_spec, row_spec, row_spec, full_spec],
        out_specs=row_spec,
        out_shape=jax.ShapeDtypeStruct((N, D), jnp.float32),
    )(r, a0, a1, w_agg)


def kernel(x, edge_index, W_src, W_dst, b_msg, W_node, W_agg, b_out):
    idx = jnp.transpose(edge_index.reshape(2, NW, NCHUNK, C),
                        (1, 2, 0, 3)).reshape(-1)
    zeros = jnp.zeros((N, D), jnp.float32)
    p, q, r = _tc_pre(x, W_src, W_dst, W_node,
                      b_msg.reshape(1, D), b_out.reshape(1, D))
    agg_part = _sc_edge(p, q, idx, zeros)
    return _tc_post(r, agg_part[0], agg_part[1], W_agg)


# speedup vs baseline: 1.0172x; 1.0172x over previous
import functools

import jax
import jax.numpy as jnp
from jax import lax
from jax.experimental import pallas as pl
from jax.experimental.pallas import tpu as pltpu
from jax.experimental.pallas import tpu_sc as plsc

N = 10000
E = 320000
D = 128

NC = 2
NS = 16
L = 16
NW = NC * NS
EPW = E // NW
C = 80
NCHUNK = EPW // C
NPAIR = NCHUNK // 2
RPT = 624
TAIL = N - NS * RPT
ZROWS = 48


_TC_BLK = 400


def _tc_pre_body(x_ref, ws_ref, wd_ref, wn_ref, bm_ref, bo_ref,
                 p_ref, q_ref, r_ref):
    xb = x_ref[...]
    p_ref[...] = jnp.dot(xb, ws_ref[...],
                         preferred_element_type=jnp.float32) + bm_ref[...]
    q_ref[...] = jnp.dot(xb, wd_ref[...], preferred_element_type=jnp.float32)
    r_ref[...] = jnp.dot(xb, wn_ref[...],
                         preferred_element_type=jnp.float32) + bo_ref[...]


def _tc_pre(x, w_src, w_dst, w_node, b_msg, b_out):
    row_spec = pl.BlockSpec((_TC_BLK, D), lambda i: (i, 0))
    full_spec = pl.BlockSpec((D, D), lambda i: (0, 0))
    bias_spec = pl.BlockSpec((1, D), lambda i: (0, 0))
    out_sds = jax.ShapeDtypeStruct((N, D), jnp.float32)
    return pl.pallas_call(
        _tc_pre_body,
        grid=(N // _TC_BLK,),
        in_specs=[row_spec, full_spec, full_spec, full_spec,
                  bias_spec, bias_spec],
        out_specs=[row_spec, row_spec, row_spec],
        out_shape=[out_sds, out_sds, out_sds],
    )(x, w_src, w_dst, w_node, b_msg, b_out)


def _sc_edge_body(p_hbm, q_hbm, idx_hbm, out_hbm,
                  sidi0, sidi1, si0, si1, di0, di1, pbuf0, pbuf1,
                  qbuf0, qbuf1, zbuf,
                  agg_sh, sem_p0, sem_p1, sem_q0, sem_q1, sem_s0, sem_s1,
                  sem_i0, sem_i1):
    c = lax.axis_index("c")
    s = lax.axis_index("s")
    wid = s * NC + c

    def zero_row(i, carry):
        for j in range(D // L):
            zbuf[i, pl.ds(j * L, L)] = jnp.zeros((L,), jnp.float32)
        return carry

    lax.fori_loop(0, ZROWS, zero_row, 0)
    for t in range(RPT // ZROWS):
        pltpu.sync_copy(zbuf, agg_sh.at[pl.ds(s * RPT + t * ZROWS, ZROWS)])

    @pl.when(s == NS - 1)
    def _zero_tail():
        pltpu.sync_copy(zbuf.at[pl.ds(0, TAIL)],
                        agg_sh.at[pl.ds(NS * RPT, TAIL)])

    plsc.subcore_barrier()

    bufs = ((sidi0, si0, di0, pbuf0, qbuf0, sem_p0, sem_q0, sem_s0, sem_i0),
            (sidi1, si1, di1, pbuf1, qbuf1, sem_p1, sem_q1, sem_s1, sem_i1))

    def idx_copy(off, b):
        sidi, _, _, _, _, _, _, _, smi = bufs[b]
        base = (wid * NCHUNK + off) * 2 * C
        pltpu.async_copy(idx_hbm.at[pl.ds(base, 2 * C)], sidi, smi)

    def wait_idx(off, b):
        sidi, _, _, _, _, _, _, _, smi = bufs[b]
        base = (wid * NCHUNK + off) * 2 * C
        pltpu.make_async_copy(idx_hbm.at[pl.ds(base, 2 * C)], sidi, smi).wait()

    def gather(b):
        sidi, si, di, pb, qb, sp, sq, _, _ = bufs[b]
        for j in range(C // L):
            si[pl.ds(j * L, L)] = sidi[pl.ds(j * L, L)]
            di[pl.ds(j * L, L)] = sidi[pl.ds(C + j * L, L)]
        pltpu.async_copy(p_hbm.at[si], pb, sp)
        pltpu.async_copy(q_hbm.at[di], qb, sq)

    def wait_gather(b):
        _, si, di, pb, qb, sp, sq, _, _ = bufs[b]
        pltpu.make_async_copy(p_hbm.at[si], pb, sp).wait()
        pltpu.make_async_copy(q_hbm.at[di], qb, sq).wait()

    def scatter(b):
        _, _, di, pb, _, _, _, ss, _ = bufs[b]
        pltpu.async_copy(pb, agg_sh.at[di], ss, add=True)

    def wait_scatter(b):
        _, _, di, pb, _, _, _, ss, _ = bufs[b]
        pltpu.make_async_copy(pb, agg_sh.at[di], ss).wait()

    def compute(b):
        _, _, _, pb, qb, _, _, _, _ = bufs[b]

        def row(e, cr):
            for j in range(D // L):
                sl = pl.ds(j * L, L)
                pb[e, sl] = jnp.maximum(pb[e, sl] + qb[e, sl], 0.0)
            return cr

        lax.fori_loop(0, C, row, 0)

    idx_copy(0, 0)
    idx_copy(1, 1)
    wait_idx(0, 0)
    gather(0)

    def pair(g, carry):
        off0 = 2 * g

        wait_gather(0)

        @pl.when(g > 0)
        def _drain1():
            wait_scatter(1)

        wait_idx(off0 + 1, 1)
        gather(1)
        compute(0)
        scatter(0)
        idx_copy(off0 + 2, 0)

        wait_gather(1)
        wait_scatter(0)

        @pl.when(g < NPAIR - 1)
        def _next0():
            wait_idx(off0 + 2, 0)
            gather(0)

        compute(1)
        scatter(1)

        @pl.when(g < NPAIR - 1)
        def _prefetch1():
            idx_copy(off0 + 3, 1)

        return carry

    lax.fori_loop(0, NPAIR, pair, 0)

    wait_idx(NCHUNK - 1, 0)
    gather(0)
    wait_gather(0)
    wait_scatter(1)
    compute(0)
    scatter(0)
    wait_scatter(0)
    plsc.subcore_barrier()

    pltpu.sync_copy(agg_sh.at[pl.ds(s * RPT, RPT)],
                    out_hbm.at[c, pl.ds(s * RPT, RPT)])

    @pl.when(s == NS - 1)
    def _copy_tail():
        pltpu.sync_copy(agg_sh.at[pl.ds(NS * RPT, TAIL)],
                        out_hbm.at[c, pl.ds(NS * RPT, TAIL)])


_sc_edge = functools.partial(
    pl.kernel,
    out_type=jax.ShapeDtypeStruct((NC, N, D), jnp.float32),
    mesh=plsc.VectorSubcoreMesh(core_axis_name="c", subcore_axis_name="s"),
    scratch_types=(
        [pltpu.VMEM((2 * C,), jnp.int32)] * 2
        + [pltpu.VMEM((C,), jnp.int32)] * 4
        + [pltpu.VMEM((C, D), jnp.float32)] * 4
        + [pltpu.VMEM((ZROWS, D), jnp.float32)]
        + [pltpu.VMEM_SHARED((N, D), jnp.float32)]
        + [pltpu.SemaphoreType.DMA] * 8
    ),
)(_sc_edge_body)


def _tc_post_body(r_ref, a0_ref, a1_ref, wa_ref, o_ref):
    agg = a0_ref[...] + a1_ref[...]
    o_ref[...] = jnp.maximum(
        r_ref[...] + jnp.dot(agg, wa_ref[...],
                             preferred_element_type=jnp.float32), 0.0)


def _tc_post(r, a0, a1, w_agg):
    row_spec = pl.BlockSpec((_TC_BLK, D), lambda i: (i, 0))
    full_spec = pl.BlockSpec((D, D), lambda i: (0, 0))
    return pl.pallas_call(
        _tc_post_body,
        grid=(N // _TC_BLK,),
        in_specs=[row_spec, row_spec, row_spec, full_spec],
        out_specs=row_spec,
        out_shape=jax.ShapeDtypeStruct((N, D), jnp.float32),
    )(r, a0, a1, w_agg)


def kernel(x, edge_index, W_src, W_dst, b_msg, W_node, W_agg, b_out):
    idx = jnp.transpose(edge_index.reshape(2, NW, NCHUNK, C),
                        (1, 2, 0, 3)).reshape(-1)
    p, q, r = _tc_pre(x, W_src, W_dst, W_node,
                      b_msg.reshape(1, D), b_out.reshape(1, D))
    agg_part = _sc_edge(p, q, idx)
    return _tc_post(r, agg_part[0], agg_part[1], W_agg)


# speedup vs baseline: 1.0942x; 1.0757x over previous
import functools

import jax
import jax.numpy as jnp
from jax import lax
from jax.experimental import pallas as pl
from jax.experimental.pallas import tpu as pltpu
from jax.experimental.pallas import tpu_sc as plsc

N = 10000
E = 320000
D = 128

NC = 2
NS = 16
L = 16
NW = NC * NS
EPW = E // NW
C = 80
NCHUNK = EPW // C
NPAIR = NCHUNK // 2
RPT = 624
TAIL = N - NS * RPT
ZROWS = 48


_TC_BLK = 400


def _tc_pre_body(x_ref, ws_ref, wd_ref, wn_ref, bm_ref, bo_ref,
                 p_ref, q_ref, r_ref):
    xb = x_ref[...]
    p_ref[...] = jnp.dot(xb, ws_ref[...],
                         preferred_element_type=jnp.float32) + bm_ref[...]
    q_ref[...] = jnp.dot(xb, wd_ref[...], preferred_element_type=jnp.float32)
    r_ref[...] = jnp.dot(xb, wn_ref[...],
                         preferred_element_type=jnp.float32) + bo_ref[...]


def _tc_pre(x, w_src, w_dst, w_node, b_msg, b_out):
    row_spec = pl.BlockSpec((_TC_BLK, D), lambda i: (i, 0))
    full_spec = pl.BlockSpec((D, D), lambda i: (0, 0))
    bias_spec = pl.BlockSpec((1, D), lambda i: (0, 0))
    out_sds = jax.ShapeDtypeStruct((N, D), jnp.float32)
    return pl.pallas_call(
        _tc_pre_body,
        grid=(N // _TC_BLK,),
        in_specs=[row_spec, full_spec, full_spec, full_spec,
                  bias_spec, bias_spec],
        out_specs=[row_spec, row_spec, row_spec],
        out_shape=[out_sds, out_sds, out_sds],
    )(x, w_src, w_dst, w_node, b_msg, b_out)


def _sc_edge_body(p_hbm, q_hbm, idx_hbm, out_hbm,
                  si0, si1, di0, di1, sc0, sc1, pbuf0, pbuf1,
                  qbuf0, qbuf1, zbuf,
                  agg_sh, sem_p0, sem_p1, sem_q0, sem_q1, sem_s0, sem_s1,
                  sem_is0, sem_is1, sem_id0, sem_id1):
    c = lax.axis_index("c")
    s = lax.axis_index("s")
    wid = s * NC + c

    def zero_row(i, carry):
        for j in range(D // L):
            zbuf[i, pl.ds(j * L, L)] = jnp.zeros((L,), jnp.float32)
        return carry

    lax.fori_loop(0, ZROWS, zero_row, 0)
    for t in range(RPT // ZROWS):
        pltpu.sync_copy(zbuf, agg_sh.at[pl.ds(s * RPT + t * ZROWS, ZROWS)])

    @pl.when(s == NS - 1)
    def _zero_tail():
        pltpu.sync_copy(zbuf.at[pl.ds(0, TAIL)],
                        agg_sh.at[pl.ds(NS * RPT, TAIL)])

    plsc.subcore_barrier()

    bufs = ((si0, di0, sc0, pbuf0, qbuf0, sem_p0, sem_q0, sem_s0,
             sem_is0, sem_id0),
            (si1, di1, sc1, pbuf1, qbuf1, sem_p1, sem_q1, sem_s1,
             sem_is1, sem_id1))

    def idx_copy(off, b):
        si, di, _, _, _, _, _, _, sis, sid = bufs[b]
        base = wid * EPW + off * C
        pltpu.async_copy(idx_hbm.at[pl.ds(base, C)], si, sis)
        pltpu.async_copy(idx_hbm.at[pl.ds(E + base, C)], di, sid)

    def wait_idx(off, b):
        si, di, _, _, _, _, _, _, sis, sid = bufs[b]
        base = wid * EPW + off * C
        pltpu.make_async_copy(idx_hbm.at[pl.ds(base, C)], si, sis).wait()
        pltpu.make_async_copy(idx_hbm.at[pl.ds(E + base, C)], di, sid).wait()

    def gather(b):
        si, di, _, pb, qb, sp, sq, _, _, _ = bufs[b]
        pltpu.async_copy(p_hbm.at[si], pb, sp)
        pltpu.async_copy(q_hbm.at[di], qb, sq)

    def wait_gather(b):
        si, di, _, pb, qb, sp, sq, _, _, _ = bufs[b]
        pltpu.make_async_copy(p_hbm.at[si], pb, sp).wait()
        pltpu.make_async_copy(q_hbm.at[di], qb, sq).wait()

    def scatter(b):
        _, _, sc, pb, _, _, _, ss, _, _ = bufs[b]
        pltpu.async_copy(pb, agg_sh.at[sc], ss, add=True)

    def wait_scatter(b):
        _, _, sc, pb, _, _, _, ss, _, _ = bufs[b]
        pltpu.make_async_copy(pb, agg_sh.at[sc], ss).wait()

    def compute(b):
        _, di, sc, pb, qb, _, _, _, _, _ = bufs[b]
        for j in range(C // L):
            sl = pl.ds(j * L, L)
            sc[sl] = di[sl]

        def row(e, cr):
            for j in range(D // L):
                sl = pl.ds(j * L, L)
                pb[e, sl] = jnp.maximum(pb[e, sl] + qb[e, sl], 0.0)
            return cr

        lax.fori_loop(0, C, row, 0)

    idx_copy(0, 0)
    idx_copy(1, 1)
    wait_idx(0, 0)
    gather(0)

    def pair(g, carry):
        off0 = 2 * g

        wait_gather(0)

        @pl.when(g > 0)
        def _drain1():
            wait_scatter(1)

        wait_idx(off0 + 1, 1)
        gather(1)
        compute(0)
        scatter(0)
        idx_copy(off0 + 2, 0)

        wait_gather(1)
        wait_scatter(0)

        @pl.when(g < NPAIR - 1)
        def _next0():
            wait_idx(off0 + 2, 0)
            gather(0)

        compute(1)
        scatter(1)

        @pl.when(g < NPAIR - 1)
        def _prefetch1():
            idx_copy(off0 + 3, 1)

        return carry

    lax.fori_loop(0, NPAIR, pair, 0)

    wait_idx(NCHUNK - 1, 0)
    gather(0)
    wait_gather(0)
    wait_scatter(1)
    compute(0)
    scatter(0)
    wait_scatter(0)
    plsc.subcore_barrier()

    pltpu.sync_copy(agg_sh.at[pl.ds(s * RPT, RPT)],
                    out_hbm.at[c, pl.ds(s * RPT, RPT)])

    @pl.when(s == NS - 1)
    def _copy_tail():
        pltpu.sync_copy(agg_sh.at[pl.ds(NS * RPT, TAIL)],
                        out_hbm.at[c, pl.ds(NS * RPT, TAIL)])


_sc_edge = functools.partial(
    pl.kernel,
    out_type=jax.ShapeDtypeStruct((NC, N, D), jnp.float32),
    mesh=plsc.VectorSubcoreMesh(core_axis_name="c", subcore_axis_name="s"),
    scratch_types=(
        [pltpu.VMEM((C,), jnp.int32)] * 6
        + [pltpu.VMEM((C, D), jnp.float32)] * 4
        + [pltpu.VMEM((ZROWS, D), jnp.float32)]
        + [pltpu.VMEM_SHARED((N, D), jnp.float32)]
        + [pltpu.SemaphoreType.DMA] * 10
    ),
)(_sc_edge_body)


def _tc_post_body(r_ref, a0_ref, a1_ref, wa_ref, o_ref):
    agg = a0_ref[...] + a1_ref[...]
    o_ref[...] = jnp.maximum(
        r_ref[...] + jnp.dot(agg, wa_ref[...],
                             preferred_element_type=jnp.float32), 0.0)


def _tc_post(r, a0, a1, w_agg):
    row_spec = pl.BlockSpec((_TC_BLK, D), lambda i: (i, 0))
    full_spec = pl.BlockSpec((D, D), lambda i: (0, 0))
    return pl.pallas_call(
        _tc_post_body,
        grid=(N // _TC_BLK,),
        in_specs=[row_spec, row_spec, row_spec, full_spec],
        out_specs=row_spec,
        out_shape=jax.ShapeDtypeStruct((N, D), jnp.float32),
    )(r, a0, a1, w_agg)


def kernel(x, edge_index, W_src, W_dst, b_msg, W_node, W_agg, b_out):
    idx = edge_index.reshape(2 * E)
    p, q, r = _tc_pre(x, W_src, W_dst, W_node,
                      b_msg.reshape(1, D), b_out.reshape(1, D))
    agg_part = _sc_edge(p, q, idx)
    return _tc_post(r, agg_part[0], agg_part[1], W_agg)
